# baseline (speedup 1.0000x reference)
import functools

import jax
import jax.numpy as jnp
from jax import lax
from jax.experimental import pallas as pl
from jax.experimental.pallas import tpu as pltpu
from jax.experimental.pallas import tpu_sc as plsc

K = 32
M = 50000
E = 800000
NTILE = 16
NW = 32
EPW = E // NW
C = 200
NCH = EPW // C
RPT = M // NTILE

assert NCH >= 3 and (NCH - 3) % 2 == 0

_mesh = plsc.VectorSubcoreMesh(core_axis_name="c", subcore_axis_name="s")

def _mul_rows(dst_v, a_v, b_v, nrows=C):
    def blk(bi, carry):
        r0 = bi * 8
        for u in range(8):
            r = r0 + u
            for h in range(K // 16):
                sl = pl.ds(h * 16, 16)
                dst_v[r, sl] = a_v[r, sl] * b_v[r, sl]
        return carry
    lax.fori_loop(0, nrows // 8, blk, 0)


def _mul_rows_scale(dst_v, a_v, b_v, s_v, nrows=C):
    def blk(bi, carry):
        d = s_v[pl.ds(bi * 16, 16)]
        for lane in range(16):
            s = d[lane]
            r = bi * 16 + lane
            for h in range(K // 16):
                sl = pl.ds(h * 16, 16)
                dst_v[r, sl] = a_v[r, sl] * b_v[r, sl] * s
        return carry
    lax.fori_loop(0, nrows // 16, blk, 0)
    rem = nrows % 16
    if rem:
        d = s_v[pl.ds(nrows - 16, 16)]
        for lane in range(16 - rem, 16):
            s = d[lane]
            r = nrows - 16 + lane
            for h in range(K // 16):
                sl = pl.ds(h * 16, 16)
                dst_v[r, sl] = a_v[r, sl] * b_v[r, sl] * s


def _edge_pass(kg0, wij_hbm, w_col, tab_hbm, dirc_hbm,
               pairs_hbm, acc_sh,
               pk_v, dir_v, w_v, g_v,
               sem_i, sem_d, sem_w, sem_g):
    scaled = dirc_hbm is not None
    base0 = kg0 * C

    def fire_loads(k, s):
        base = base0 + k * C
        pltpu.async_copy(pairs_hbm.at[kg0 + k], pk_v[s], sem_i[s])
        pltpu.async_copy(wij_hbm.at[pl.ds(base, C), pl.ds(w_col, K)],
                         w_v[s], sem_w[s])
        if scaled:
            pltpu.async_copy(dirc_hbm.at[pl.ds(base, C)], dir_v[s], sem_d[s])

    def prep_gather(k, s):
        pltpu.make_async_copy(pairs_hbm.at[kg0 + k], pk_v[s],
                              sem_i[s]).wait()
        pltpu.async_copy(tab_hbm.at[pk_v[s].at[1]], g_v[s], sem_g[s])

    def do_chunk(k, s):
        base = base0 + k * C
        pltpu.make_async_copy(wij_hbm.at[pl.ds(base, C), pl.ds(w_col, K)],
                              w_v[s], sem_w[s]).wait()
        if scaled:
            pltpu.make_async_copy(dirc_hbm.at[pl.ds(base, C)], dir_v[s],
                                  sem_d[s]).wait()
        pltpu.make_async_copy(tab_hbm.at[pk_v[s].at[1]], g_v[s],
                              sem_g[s]).wait()
        if scaled:
            _mul_rows_scale(w_v[s], w_v[s], g_v[s], dir_v[s])
        else:
            _mul_rows(w_v[s], w_v[s], g_v[s])
        pltpu.sync_copy(w_v[s], acc_sh.at[pk_v[s].at[0]], add=True)

    fire_loads(0, 0)
    prep_gather(0, 0)
    fire_loads(1, 1)

    def body(i, carry):
        for u in range(2):
            k = 2 * i + u
            s, s2 = u, 1 - u
            prep_gather(k + 1, s2)
            do_chunk(k, s)
            fire_loads(k + 2, s)
        return carry
    lax.fori_loop(0, (NCH - 3) // 2, body, 0)

    prep_gather(NCH - 2, 1)
    do_chunk(NCH - 3, 0)
    fire_loads(NCH - 1, 0)
    prep_gather(NCH - 1, 0)
    do_chunk(NCH - 2, 1)
    do_chunk(NCH - 1, 0)


def _sc_scratch(with_dir):
    sc = [
        pltpu.VMEM((2, C), jnp.int32), pltpu.VMEM((2, C), jnp.int32),
    ]
    if with_dir:
        sc += [pltpu.VMEM((C,), jnp.float32), pltpu.VMEM((C,), jnp.float32)]
    sc += [
        pltpu.VMEM((C, K), jnp.float32), pltpu.VMEM((C, K), jnp.float32),
        pltpu.VMEM((C, K), jnp.float32), pltpu.VMEM((C, K), jnp.float32),
        pltpu.VMEM_SHARED((M, K), jnp.float32),
    ]
    sc += [pltpu.SemaphoreType.DMA] * (8 if with_dir else 6)
    return sc


@functools.partial(
    pl.kernel,
    out_type=jax.ShapeDtypeStruct((2, M, K), jnp.float32),
    mesh=_mesh,
    compiler_params=pltpu.CompilerParams(use_tc_tiling_on_sc=False),
    scratch_types=_sc_scratch(with_dir=False),
)
def _dq_kernel(wij_hbm, pairs_hbm, xq_hbm, zeros_hbm, out_hbm,
               pk0, pk1, w0, w1, g0, g1, acc_sh,
               si0, si1, sw0, sw1, sg0, sg1):
    cid = lax.axis_index("c")
    sid = lax.axis_index("s")
    wid = cid * NTILE + sid
    pltpu.sync_copy(zeros_hbm, acc_sh.at[pl.ds(sid * RPT, RPT)])
    plsc.subcore_barrier()
    _edge_pass(wid * NCH, wij_hbm, 0, xq_hbm, None, pairs_hbm,
               acc_sh, [pk0, pk1], None, [w0, w1], [g0, g1],
               [si0, si1], None, [sw0, sw1], [sg0, sg1])
    plsc.subcore_barrier()
    rows = pl.ds(sid * RPT, RPT)
    pltpu.sync_copy(acc_sh.at[rows], out_hbm.at[cid, rows])


@functools.partial(
    pl.kernel,
    out_type=jax.ShapeDtypeStruct((2, M, K), jnp.float32),
    mesh=_mesh,
    compiler_params=pltpu.CompilerParams(use_tc_tiling_on_sc=False),
    scratch_types=_sc_scratch(with_dir=True),
)
def _dmu_kernel(wij_hbm, pairs_hbm, dirc_hbm, xr_hbm, tc_hbm,
                zeros_hbm, out_hbm,
                pk0, pk1, d0, d1, w0, w1, g0, g1, acc_sh,
                si0, si1, sd0, sd1, sw0, sw1, sg0, sg1):
    cid = lax.axis_index("c")
    sid = lax.axis_index("s")
    wid = cid * NTILE + sid
    pltpu.sync_copy(zeros_hbm, acc_sh.at[pl.ds(sid * RPT, RPT)])
    plsc.subcore_barrier()
    kg0 = wid * NCH
    _edge_pass(kg0, wij_hbm, K, xr_hbm, dirc_hbm, pairs_hbm,
               acc_sh, [pk0, pk1], [d0, d1], [w0, w1], [g0, g1],
               [si0, si1], [sd0, sd1], [sw0, sw1], [sg0, sg1])
    _edge_pass(kg0, wij_hbm, 2 * K, tc_hbm, None, pairs_hbm,
               acc_sh, [pk0, pk1], None, [w0, w1], [g0, g1],
               [si0, si1], None, [sw0, sw1], [sg0, sg1])
    plsc.subcore_barrier()
    rows = pl.ds(sid * RPT, RPT)
    pltpu.sync_copy(acc_sh.at[rows], out_hbm.at[cid, rows])


_RB = 2000


def _mlp_body(emb_ref, mu_ref, w1_ref, b1_ref, w2_ref, b2_ref,
              xq_ref, xr_ref, t0_ref, t1_ref, t2_ref):
    h = jnp.dot(emb_ref[...], w1_ref[...], preferred_element_type=jnp.float32)
    h = h + b1_ref[...]
    h = h * lax.logistic(h)
    x = jnp.dot(h, w2_ref[...], preferred_element_type=jnp.float32)
    x = x + b2_ref[...]
    xq_ref[...] = x[:, 0:K]
    xr_ref[...] = x[:, K:2 * K]
    xm = x[:, 2 * K:3 * K]
    t0_ref[...] = xm * mu_ref[:, 0:K]
    t1_ref[...] = xm * mu_ref[:, K:2 * K]
    t2_ref[...] = xm * mu_ref[:, 2 * K:3 * K]


def _combine_body(emb_ref, mu_ref, dq_ref, d0_ref, d1_ref, d2_ref,
                  q_ref, mo_ref):
    q_ref[...] = emb_ref[...] + dq_ref[0] + dq_ref[1]
    dmu = jnp.concatenate(
        [d0_ref[0] + d0_ref[1], d1_ref[0] + d1_ref[1], d2_ref[0] + d2_ref[1]],
        axis=-1)
    mo_ref[...] = mu_ref[...] + dmu


def kernel(atomic_numbers_embedding, mu, Wij, dir_ij, pairlist, n_atoms,
           W1, b1, W2, b2):
    del n_atoms
    n, m, k = atomic_numbers_embedding.shape
    emb2d = atomic_numbers_embedding.reshape(m, k)
    wij2d = Wij.reshape(E, 3 * K)
    mu96 = mu.reshape(M, 3 * K)

    xq, xr, t0, t1, t2 = pl.pallas_call(
        _mlp_body,
        grid=(M // _RB,),
        in_specs=[
            pl.BlockSpec((_RB, K), lambda i: (i, 0)),
            pl.BlockSpec((_RB, 3 * K), lambda i: (i, 0)),
            pl.BlockSpec((K, K), lambda i: (0, 0)),
            pl.BlockSpec((1, K), lambda i: (0, 0)),
            pl.BlockSpec((K, 3 * K), lambda i: (0, 0)),
            pl.BlockSpec((1, 3 * K), lambda i: (0, 0)),
        ],
        out_specs=[pl.BlockSpec((_RB, K), lambda i: (i, 0))] * 5,
        out_shape=[jax.ShapeDtypeStruct((M, K), jnp.float32)] * 5,
    )(emb2d, mu96, W1, b1.reshape(1, K), W2, b2.reshape(1, 3 * K))

    zeros = jnp.zeros((RPT, K), jnp.float32)
    dir_t = dir_ij.T
    pairs = pairlist.reshape(2, NW * NCH, C).transpose(1, 0, 2)

    dq = _dq_kernel(wij2d, pairs, xq, zeros)
    tabs = (t0, t1, t2)
    dmu_parts = []
    for c in range(3):
        dmu_parts.append(
            _dmu_kernel(wij2d, pairs, dir_t[c], xr, tabs[c], zeros))

    q2d, mo96 = pl.pallas_call(
        _combine_body,
        grid=(M // _RB,),
        in_specs=[
            pl.BlockSpec((_RB, K), lambda i: (i, 0)),
            pl.BlockSpec((_RB, 3 * K), lambda i: (i, 0)),
            pl.BlockSpec((2, _RB, K), lambda i: (0, i, 0)),
            pl.BlockSpec((2, _RB, K), lambda i: (0, i, 0)),
            pl.BlockSpec((2, _RB, K), lambda i: (0, i, 0)),
            pl.BlockSpec((2, _RB, K), lambda i: (0, i, 0)),
        ],
        out_specs=[
            pl.BlockSpec((_RB, K), lambda i: (i, 0)),
            pl.BlockSpec((_RB, 3 * K), lambda i: (i, 0)),
        ],
        out_shape=[
            jax.ShapeDtypeStruct((M, K), jnp.float32),
            jax.ShapeDtypeStruct((M, 3 * K), jnp.float32),
        ],
    )(emb2d, mu96, dq, dmu_parts[0], dmu_parts[1], dmu_parts[2])

    return (q2d.reshape(n, m, k), mo96.reshape(M, 3, K))


# speedup vs baseline: 1.0211x; 1.0211x over previous
import functools

import jax
import jax.numpy as jnp
from jax import lax
from jax.experimental import pallas as pl
from jax.experimental.pallas import tpu as pltpu
from jax.experimental.pallas import tpu_sc as plsc

K = 32
M = 50000
E = 800000
NTILE = 16
NW = 32
EPW = E // NW
C = 200
NCH = EPW // C
RPT = M // NTILE

assert NCH >= 3 and (NCH - 3) % 2 == 0

_mesh = plsc.VectorSubcoreMesh(core_axis_name="c", subcore_axis_name="s")

def _mul_rows(dst_v, a_v, b_v, nrows=C):
    def blk(bi, carry):
        r0 = bi * 8
        for u in range(8):
            r = r0 + u
            for h in range(K // 16):
                sl = pl.ds(h * 16, 16)
                dst_v[r, sl] = a_v[r, sl] * b_v[r, sl]
        return carry
    lax.fori_loop(0, nrows // 8, blk, 0)


def _mul_rows_scale(dst_v, a_v, b_v, s_v, nrows=C):
    def blk(bi, carry):
        d = s_v[pl.ds(bi * 16, 16)]
        for lane in range(16):
            s = d[lane]
            r = bi * 16 + lane
            for h in range(K // 16):
                sl = pl.ds(h * 16, 16)
                dst_v[r, sl] = a_v[r, sl] * b_v[r, sl] * s
        return carry
    lax.fori_loop(0, nrows // 16, blk, 0)
    rem = nrows % 16
    if rem:
        d = s_v[pl.ds(nrows - 16, 16)]
        for lane in range(16 - rem, 16):
            s = d[lane]
            r = nrows - 16 + lane
            for h in range(K // 16):
                sl = pl.ds(h * 16, 16)
                dst_v[r, sl] = a_v[r, sl] * b_v[r, sl] * s


def _edge_pass(base0, wij_hbm, w_col, tab_hbm, dirc_hbm,
               idxi_hbm, idxj_hbm, acc_sh,
               idxi_v, idxj_v, dir_v, w_v, g_v,
               sem_i, sem_j, sem_d, sem_w, sem_g):
    scaled = dirc_hbm is not None

    def fire_loads(k, s):
        base = base0 + k * C
        pltpu.async_copy(idxi_hbm.at[pl.ds(base, C)], idxi_v[s], sem_i[s])
        pltpu.async_copy(idxj_hbm.at[pl.ds(base, C)], idxj_v[s], sem_j[s])
        pltpu.async_copy(wij_hbm.at[pl.ds(base, C), pl.ds(w_col, K)],
                         w_v[s], sem_w[s])
        if scaled:
            pltpu.async_copy(dirc_hbm.at[pl.ds(base, C)], dir_v[s], sem_d[s])

    def prep_gather(k, s):
        base = base0 + k * C
        pltpu.make_async_copy(idxj_hbm.at[pl.ds(base, C)], idxj_v[s],
                              sem_j[s]).wait()
        pltpu.async_copy(tab_hbm.at[idxj_v[s]], g_v[s], sem_g[s])

    def do_chunk(k, s):
        base = base0 + k * C
        pltpu.make_async_copy(idxi_hbm.at[pl.ds(base, C)], idxi_v[s],
                              sem_i[s]).wait()
        pltpu.make_async_copy(wij_hbm.at[pl.ds(base, C), pl.ds(w_col, K)],
                              w_v[s], sem_w[s]).wait()
        if scaled:
            pltpu.make_async_copy(dirc_hbm.at[pl.ds(base, C)], dir_v[s],
                                  sem_d[s]).wait()
        pltpu.make_async_copy(tab_hbm.at[idxj_v[s]], g_v[s], sem_g[s]).wait()
        if scaled:
            _mul_rows_scale(w_v[s], w_v[s], g_v[s], dir_v[s])
        else:
            _mul_rows(w_v[s], w_v[s], g_v[s])
        pltpu.sync_copy(w_v[s], acc_sh.at[idxi_v[s]], add=True)

    fire_loads(0, 0)
    prep_gather(0, 0)
    fire_loads(1, 1)

    def body(i, carry):
        for u in range(2):
            k = 2 * i + u
            s, s2 = u, 1 - u
            prep_gather(k + 1, s2)
            do_chunk(k, s)
            fire_loads(k + 2, s)
        return carry
    lax.fori_loop(0, (NCH - 3) // 2, body, 0)

    prep_gather(NCH - 2, 1)
    do_chunk(NCH - 3, 0)
    fire_loads(NCH - 1, 0)
    prep_gather(NCH - 1, 0)
    do_chunk(NCH - 2, 1)
    do_chunk(NCH - 1, 0)


def _sc_scratch(with_dir):
    sc = [
        pltpu.VMEM((C,), jnp.int32), pltpu.VMEM((C,), jnp.int32),
        pltpu.VMEM((C,), jnp.int32), pltpu.VMEM((C,), jnp.int32),
    ]
    if with_dir:
        sc += [pltpu.VMEM((C,), jnp.float32), pltpu.VMEM((C,), jnp.float32)]
    sc += [
        pltpu.VMEM((C, K), jnp.float32), pltpu.VMEM((C, K), jnp.float32),
        pltpu.VMEM((C, K), jnp.float32), pltpu.VMEM((C, K), jnp.float32),
        pltpu.VMEM_SHARED((M, K), jnp.float32),
    ]
    sc += [pltpu.SemaphoreType.DMA] * (10 if with_dir else 8)
    return sc


@functools.partial(
    pl.kernel,
    out_type=jax.ShapeDtypeStruct((2, M, K), jnp.float32),
    mesh=_mesh,
    compiler_params=pltpu.CompilerParams(use_tc_tiling_on_sc=False),
    scratch_types=_sc_scratch(with_dir=False),
)
def _dq_kernel(wij_hbm, idxi_hbm, idxj_hbm, xq_hbm, zeros_hbm, out_hbm,
               ii0, ii1, ij0, ij1, w0, w1, g0, g1, acc_sh,
               si0, si1, sj0, sj1, sw0, sw1, sg0, sg1):
    cid = lax.axis_index("c")
    sid = lax.axis_index("s")
    wid = cid * NTILE + sid
    pltpu.sync_copy(zeros_hbm, acc_sh.at[pl.ds(sid * RPT, RPT)])
    plsc.subcore_barrier()
    _edge_pass(wid * EPW, wij_hbm, 0, xq_hbm, None, idxi_hbm, idxj_hbm,
               acc_sh, [ii0, ii1], [ij0, ij1], None, [w0, w1], [g0, g1],
               [si0, si1], [sj0, sj1], None, [sw0, sw1], [sg0, sg1])
    plsc.subcore_barrier()
    rows = pl.ds(sid * RPT, RPT)
    pltpu.sync_copy(acc_sh.at[rows], out_hbm.at[cid, rows])


@functools.partial(
    pl.kernel,
    out_type=jax.ShapeDtypeStruct((2, M, K), jnp.float32),
    mesh=_mesh,
    compiler_params=pltpu.CompilerParams(use_tc_tiling_on_sc=False),
    scratch_types=_sc_scratch(with_dir=True),
)
def _dmu_kernel(wij_hbm, idxi_hbm, idxj_hbm, dirc_hbm, xr_hbm, tc_hbm,
                zeros_hbm, out_hbm,
                ii0, ii1, ij0, ij1, d0, d1, w0, w1, g0, g1, acc_sh,
                si0, si1, sj0, sj1, sd0, sd1, sw0, sw1, sg0, sg1):
    cid = lax.axis_index("c")
    sid = lax.axis_index("s")
    wid = cid * NTILE + sid
    pltpu.sync_copy(zeros_hbm, acc_sh.at[pl.ds(sid * RPT, RPT)])
    plsc.subcore_barrier()
    base0 = wid * EPW
    _edge_pass(base0, wij_hbm, K, xr_hbm, dirc_hbm, idxi_hbm, idxj_hbm,
               acc_sh, [ii0, ii1], [ij0, ij1], [d0, d1], [w0, w1], [g0, g1],
               [si0, si1], [sj0, sj1], [sd0, sd1], [sw0, sw1], [sg0, sg1])
    _edge_pass(base0, wij_hbm, 2 * K, tc_hbm, None, idxi_hbm, idxj_hbm,
               acc_sh, [ii0, ii1], [ij0, ij1], None, [w0, w1], [g0, g1],
               [si0, si1], [sj0, sj1], None, [sw0, sw1], [sg0, sg1])
    plsc.subcore_barrier()
    rows = pl.ds(sid * RPT, RPT)
    pltpu.sync_copy(acc_sh.at[rows], out_hbm.at[cid, rows])


_RB = 2000


def _mlp_body(emb_ref, mu_ref, w1_ref, b1_ref, w2_ref, b2_ref,
              xq_ref, xr_ref, t0_ref, t1_ref, t2_ref):
    h = jnp.dot(emb_ref[...], w1_ref[...], preferred_element_type=jnp.float32)
    h = h + b1_ref[...]
    h = h * lax.logistic(h)
    x = jnp.dot(h, w2_ref[...], preferred_element_type=jnp.float32)
    x = x + b2_ref[...]
    xq_ref[...] = x[:, 0:K]
    xr_ref[...] = x[:, K:2 * K]
    xm = x[:, 2 * K:3 * K]
    t0_ref[...] = xm * mu_ref[:, 0:K]
    t1_ref[...] = xm * mu_ref[:, K:2 * K]
    t2_ref[...] = xm * mu_ref[:, 2 * K:3 * K]


def _combine_body(emb_ref, mu_ref, dq_ref, d0_ref, d1_ref, d2_ref,
                  q_ref, mo_ref):
    q_ref[...] = emb_ref[...] + dq_ref[0] + dq_ref[1]
    dmu = jnp.concatenate(
        [d0_ref[0] + d0_ref[1], d1_ref[0] + d1_ref[1], d2_ref[0] + d2_ref[1]],
        axis=-1)
    mo_ref[...] = mu_ref[...] + dmu


def kernel(atomic_numbers_embedding, mu, Wij, dir_ij, pairlist, n_atoms,
           W1, b1, W2, b2):
    del n_atoms
    n, m, k = atomic_numbers_embedding.shape
    emb2d = atomic_numbers_embedding.reshape(m, k)
    wij2d = Wij.reshape(E, 3 * K)
    mu96 = mu.reshape(M, 3 * K)

    xq, xr, t0, t1, t2 = pl.pallas_call(
        _mlp_body,
        grid=(M // _RB,),
        in_specs=[
            pl.BlockSpec((_RB, K), lambda i: (i, 0)),
            pl.BlockSpec((_RB, 3 * K), lambda i: (i, 0)),
            pl.BlockSpec((K, K), lambda i: (0, 0)),
            pl.BlockSpec((1, K), lambda i: (0, 0)),
            pl.BlockSpec((K, 3 * K), lambda i: (0, 0)),
            pl.BlockSpec((1, 3 * K), lambda i: (0, 0)),
        ],
        out_specs=[pl.BlockSpec((_RB, K), lambda i: (i, 0))] * 5,
        out_shape=[jax.ShapeDtypeStruct((M, K), jnp.float32)] * 5,
    )(emb2d, mu96, W1, b1.reshape(1, K), W2, b2.reshape(1, 3 * K))

    zeros = jnp.zeros((RPT, K), jnp.float32)
    dir_t = dir_ij.T
    idx_i = pairlist[0]
    idx_j = pairlist[1]

    dq = _dq_kernel(wij2d, idx_i, idx_j, xq, zeros)
    tabs = (t0, t1, t2)
    dmu_parts = []
    for c in range(3):
        dmu_parts.append(
            _dmu_kernel(wij2d, idx_i, idx_j, dir_t[c], xr, tabs[c], zeros))

    q2d, mo96 = pl.pallas_call(
        _combine_body,
        grid=(M // _RB,),
        in_specs=[
            pl.BlockSpec((_RB, K), lambda i: (i, 0)),
            pl.BlockSpec((_RB, 3 * K), lambda i: (i, 0)),
            pl.BlockSpec((2, _RB, K), lambda i: (0, i, 0)),
            pl.BlockSpec((2, _RB, K), lambda i: (0, i, 0)),
            pl.BlockSpec((2, _RB, K), lambda i: (0, i, 0)),
            pl.BlockSpec((2, _RB, K), lambda i: (0, i, 0)),
        ],
        out_specs=[
            pl.BlockSpec((_RB, K), lambda i: (i, 0)),
            pl.BlockSpec((_RB, 3 * K), lambda i: (i, 0)),
        ],
        out_shape=[
            jax.ShapeDtypeStruct((M, K), jnp.float32),
            jax.ShapeDtypeStruct((M, 3 * K), jnp.float32),
        ],
    )(emb2d, mu96, dq, dmu_parts[0], dmu_parts[1], dmu_parts[2])

    return (q2d.reshape(n, m, k), mo96.reshape(M, 3, K))
